# Initial kernel scaffold; baseline (speedup 1.0000x reference)
import jax
import jax.numpy as jnp
from jax.experimental import pallas as pl


def kernel(x, edge_index, W0, b0, W1, b1, Wc, bc):
    raise NotImplementedError("write your pallas kernel here")


# speedup vs baseline: 8.5889x; 8.5889x over previous
import functools

import jax
import jax.numpy as jnp
from jax import lax
from jax.experimental import pallas as pl
from jax.experimental.pallas import tpu as pltpu
from jax.experimental.pallas import tpu_sc as plsc

N = 10000
E = 320000
D = 128
NC_OUT = 40

N_PAD = 10240
CH = 128
NCORES = 2
NSUB = 16
NW = NCORES * NSUB
CPT = 80
E_PAD = NW * CPT * CH
ROWS_PER_TILE = N_PAD // NSUB


def _sc_mesh():
    return plsc.VectorSubcoreMesh(core_axis_name="c", subcore_axis_name="s")


def _deg_kernel_body(dst_hbm, zeros_hbm, out_hbm, dst_v, ones_v, accd, sem):
    cid = lax.axis_index("c")
    sid = lax.axis_index("s")
    tile = cid * NSUB + sid

    def fill_ones(i, c):
        ones_v[i] = jnp.ones((16,), jnp.float32)
        return c
    lax.fori_loop(0, CH, fill_ones, 0)

    r0 = sid * ROWS_PER_TILE
    pltpu.sync_copy(zeros_hbm.at[pl.ds(r0, ROWS_PER_TILE)],
                    accd.at[pl.ds(r0, ROWS_PER_TILE)])
    plsc.subcore_barrier()

    base = tile * CPT
    pltpu.sync_copy(dst_hbm.at[pl.ds(base, CPT)], dst_v)

    def body(j, c):
        pltpu.sync_copy(ones_v, accd.at[dst_v.at[j]], add=True)
        return c
    lax.fori_loop(0, CPT, body, 0)
    plsc.subcore_barrier()

    pltpu.sync_copy(accd.at[pl.ds(r0, ROWS_PER_TILE)],
                    out_hbm.at[cid, pl.ds(r0, ROWS_PER_TILE)])


def _make_deg_kernel():
    return functools.partial(
        pl.kernel,
        mesh=_sc_mesh(),
        out_type=jax.ShapeDtypeStruct((NCORES, N_PAD, 16), jnp.float32),
        scratch_types=[
            pltpu.VMEM((CPT, CH), jnp.int32),
            pltpu.VMEM((CH, 16), jnp.float32),
            pltpu.VMEM_SHARED((N_PAD, 16), jnp.float32),
            pltpu.SemaphoreType.DMA,
        ],
    )(_deg_kernel_body)


def _scat_kernel_body(z_hbm, src_hbm, dst_hbm, zeros_hbm, out_hbm,
                      src_v, dst_v, rows, accs, sem):
    cid = lax.axis_index("c")
    sid = lax.axis_index("s")
    tile = cid * NSUB + sid

    r0 = sid * ROWS_PER_TILE
    pltpu.sync_copy(zeros_hbm.at[pl.ds(r0, ROWS_PER_TILE)],
                    accs.at[pl.ds(r0, ROWS_PER_TILE)])
    plsc.subcore_barrier()

    base = tile * CPT
    pltpu.sync_copy(src_hbm.at[pl.ds(base, CPT)], src_v)
    pltpu.sync_copy(dst_hbm.at[pl.ds(base, CPT)], dst_v)

    def body(j, c):
        pltpu.async_copy(z_hbm.at[src_v.at[j]], rows, sem).wait()
        pltpu.sync_copy(rows, accs.at[dst_v.at[j]], add=True)
        return c
    lax.fori_loop(0, CPT, body, 0)
    plsc.subcore_barrier()

    pltpu.sync_copy(accs.at[pl.ds(r0, ROWS_PER_TILE)],
                    out_hbm.at[cid, pl.ds(r0, ROWS_PER_TILE)])


def _make_scat_kernel():
    return functools.partial(
        pl.kernel,
        mesh=_sc_mesh(),
        out_type=jax.ShapeDtypeStruct((NCORES, N_PAD, D), jnp.float32),
        scratch_types=[
            pltpu.VMEM((CPT, CH), jnp.int32),
            pltpu.VMEM((CPT, CH), jnp.int32),
            pltpu.VMEM((CH, D), jnp.float32),
            pltpu.VMEM_SHARED((N_PAD, D), jnp.float32),
            pltpu.SemaphoreType.DMA,
        ],
    )(_scat_kernel_body)


_BLK = 1024


def _tc_b_body(x_ref, w_ref, degp_ref, z_ref, d_ref):
    deg = 1.0 + degp_ref[0, :, 0] + degp_ref[1, :, 0]
    d = lax.rsqrt(deg)
    z = jnp.dot(x_ref[...], w_ref[...], preferred_element_type=jnp.float32)
    z_ref[...] = z * d[:, None]
    d_ref[...] = d[:, None]


def _tc_b(x_pad, W0, degp):
    grid = (N_PAD // _BLK,)
    return pl.pallas_call(
        _tc_b_body,
        grid=grid,
        in_specs=[
            pl.BlockSpec((_BLK, D), lambda i: (i, 0)),
            pl.BlockSpec((D, D), lambda i: (0, 0)),
            pl.BlockSpec((NCORES, _BLK, 16), lambda i: (0, i, 0)),
        ],
        out_specs=[
            pl.BlockSpec((_BLK, D), lambda i: (i, 0)),
            pl.BlockSpec((_BLK, 1), lambda i: (i, 0)),
        ],
        out_shape=[
            jax.ShapeDtypeStruct((N_PAD, D), jnp.float32),
            jax.ShapeDtypeStruct((N_PAD, 1), jnp.float32),
        ],
    )(x_pad, W0, degp)


def _tc_d_body(z_ref, p_ref, d_ref, b_ref, w_ref, z1_ref):
    s = p_ref[0] + p_ref[1] + z_ref[...]
    h = jnp.maximum(d_ref[...] * s + b_ref[...], 0.0)
    z1 = jnp.dot(h, w_ref[...], preferred_element_type=jnp.float32)
    z1_ref[...] = z1 * d_ref[...]


def _tc_d(z0, p, d, b0, W1):
    grid = (N_PAD // _BLK,)
    return pl.pallas_call(
        _tc_d_body,
        grid=grid,
        in_specs=[
            pl.BlockSpec((_BLK, D), lambda i: (i, 0)),
            pl.BlockSpec((NCORES, _BLK, D), lambda i: (0, i, 0)),
            pl.BlockSpec((_BLK, 1), lambda i: (i, 0)),
            pl.BlockSpec((1, D), lambda i: (0, 0)),
            pl.BlockSpec((D, D), lambda i: (0, 0)),
        ],
        out_specs=pl.BlockSpec((_BLK, D), lambda i: (i, 0)),
        out_shape=jax.ShapeDtypeStruct((N_PAD, D), jnp.float32),
    )(z0, p, d, b0, W1)


def _tc_e_body(z_ref, p_ref, d_ref, b_ref, wc_ref, bc_ref, o_ref):
    s = p_ref[0] + p_ref[1] + z_ref[...]
    h = jnp.maximum(d_ref[...] * s + b_ref[...], 0.0)
    logits = jnp.dot(h, wc_ref[...], preferred_element_type=jnp.float32)
    logits = logits + bc_ref[...]
    m = jnp.max(logits, axis=1, keepdims=True)
    ex = jnp.exp(logits - m)
    lse = jnp.log(jnp.sum(ex, axis=1, keepdims=True)) + m
    o_ref[...] = logits - lse


def _tc_e(z1, p, d, b1, Wc_pad, bc_pad):
    grid = (N_PAD // _BLK,)
    return pl.pallas_call(
        _tc_e_body,
        grid=grid,
        in_specs=[
            pl.BlockSpec((_BLK, D), lambda i: (i, 0)),
            pl.BlockSpec((NCORES, _BLK, D), lambda i: (0, i, 0)),
            pl.BlockSpec((_BLK, 1), lambda i: (i, 0)),
            pl.BlockSpec((1, D), lambda i: (0, 0)),
            pl.BlockSpec((D, D), lambda i: (0, 0)),
            pl.BlockSpec((1, D), lambda i: (0, 0)),
        ],
        out_specs=pl.BlockSpec((_BLK, D), lambda i: (i, 0)),
        out_shape=jax.ShapeDtypeStruct((N_PAD, D), jnp.float32),
    )(z1, p, d, b1, Wc_pad, bc_pad)


def kernel(x, edge_index, W0, b0, W1, b1, Wc, bc):
    src = edge_index[0]
    dst = edge_index[1]
    pad = E_PAD - src.shape[0]
    src_p = jnp.concatenate([src, jnp.full((pad,), N, jnp.int32)])
    dst_p = jnp.concatenate([dst, jnp.full((pad,), N, jnp.int32)])
    src2d = src_p.reshape(NW * CPT, CH)
    dst2d = dst_p.reshape(NW * CPT, CH)

    x_pad = jnp.zeros((N_PAD, D), jnp.float32).at[:N].set(x)
    zeros16 = jnp.zeros((N_PAD, 16), jnp.float32)
    zerosD = jnp.zeros((N_PAD, D), jnp.float32)
    b0r = b0.reshape(1, D)
    b1r = b1.reshape(1, D)
    Wc_pad = jnp.zeros((D, D), jnp.float32).at[:, :NC_OUT].set(Wc)
    bc_pad = jnp.full((1, D), -1e30, jnp.float32).at[0, :NC_OUT].set(bc)

    deg_k = _make_deg_kernel()
    scat_k = _make_scat_kernel()

    degp = deg_k(dst2d, zeros16)
    z0, d = _tc_b(x_pad, W0, degp)
    p0 = scat_k(z0, src2d, dst2d, zerosD)
    z1 = _tc_d(z0, p0, d, b0r, W1)
    p1 = scat_k(z1, src2d, dst2d, zerosD)
    full = _tc_e(z1, p1, d, b1r, Wc_pad, bc_pad)
    return full[:N, :NC_OUT]


# speedup vs baseline: 21.3107x; 2.4812x over previous
import functools

import jax
import jax.numpy as jnp
from jax import lax
from jax.experimental import pallas as pl
from jax.experimental.pallas import tpu as pltpu
from jax.experimental.pallas import tpu_sc as plsc

N = 10000
E = 320000
D = 128
NC_OUT = 40

N_PAD = 10240
CH = 128
NCORES = 2
NSUB = 16
NW = NCORES * NSUB
CPT = 80
E_PAD = NW * CPT * CH
ROWS_PER_TILE = N_PAD // NSUB


def _sc_mesh():
    return plsc.VectorSubcoreMesh(core_axis_name="c", subcore_axis_name="s")


def _deg_kernel_body(dst_hbm, zeros_hbm, out_hbm, dst_v, ones_v, accd, sem):
    cid = lax.axis_index("c")
    sid = lax.axis_index("s")
    tile = cid * NSUB + sid

    def fill_ones(i, c):
        ones_v[i] = jnp.ones((16,), jnp.float32)
        return c
    lax.fori_loop(0, CH, fill_ones, 0)

    r0 = sid * ROWS_PER_TILE
    pltpu.sync_copy(zeros_hbm.at[pl.ds(r0, ROWS_PER_TILE)],
                    accd.at[pl.ds(r0, ROWS_PER_TILE)])
    plsc.subcore_barrier()

    base = tile * CPT
    pltpu.sync_copy(dst_hbm.at[pl.ds(base, CPT)], dst_v)

    def body(j, c):
        pltpu.sync_copy(ones_v, accd.at[dst_v.at[j]], add=True)
        return c
    lax.fori_loop(0, CPT, body, 0)
    plsc.subcore_barrier()

    pltpu.sync_copy(accd.at[pl.ds(r0, ROWS_PER_TILE)],
                    out_hbm.at[cid, pl.ds(r0, ROWS_PER_TILE)])


def _make_deg_kernel():
    return functools.partial(
        pl.kernel,
        mesh=_sc_mesh(),
        out_type=jax.ShapeDtypeStruct((NCORES, N_PAD, 16), jnp.float32),
        scratch_types=[
            pltpu.VMEM((CPT, CH), jnp.int32),
            pltpu.VMEM((CH, 16), jnp.float32),
            pltpu.VMEM_SHARED((N_PAD, 16), jnp.float32),
            pltpu.SemaphoreType.DMA,
        ],
    )(_deg_kernel_body)


def _scat_kernel_body(z_hbm, src_hbm, dst_hbm, zeros_hbm, out_hbm,
                      src_v, dst_v, rows, accs, sem):
    cid = lax.axis_index("c")
    sid = lax.axis_index("s")
    tile = cid * NSUB + sid

    r0 = sid * ROWS_PER_TILE
    pltpu.sync_copy(zeros_hbm.at[pl.ds(r0, ROWS_PER_TILE)],
                    accs.at[pl.ds(r0, ROWS_PER_TILE)])
    plsc.subcore_barrier()

    base = tile * CPT
    pltpu.sync_copy(src_hbm.at[pl.ds(base, CPT)], src_v)
    pltpu.sync_copy(dst_hbm.at[pl.ds(base, CPT)], dst_v)

    def body(j, c):
        pltpu.async_copy(z_hbm.at[src_v.at[j]], rows, sem).wait()
        pltpu.sync_copy(rows, accs.at[dst_v.at[j]], add=True)
        return c
    lax.fori_loop(0, CPT, body, 0)
    plsc.subcore_barrier()

    pltpu.sync_copy(accs.at[pl.ds(r0, ROWS_PER_TILE)],
                    out_hbm.at[cid, pl.ds(r0, ROWS_PER_TILE)])


def _make_scat_kernel():
    return functools.partial(
        pl.kernel,
        mesh=_sc_mesh(),
        out_type=jax.ShapeDtypeStruct((NCORES, N_PAD, D), jnp.float32),
        scratch_types=[
            pltpu.VMEM((CPT, CH), jnp.int32),
            pltpu.VMEM((CPT, CH), jnp.int32),
            pltpu.VMEM((CH, D), jnp.float32),
            pltpu.VMEM_SHARED((N_PAD, D), jnp.float32),
            pltpu.SemaphoreType.DMA,
        ],
    )(_scat_kernel_body)


_BLK = 1024


def _tc_b_body(x_ref, w_ref, degp_ref, z_ref, d_ref):
    deg = 1.0 + degp_ref[0, :, 0] + degp_ref[1, :, 0]
    d = lax.rsqrt(deg)
    z = jnp.dot(x_ref[...], w_ref[...], preferred_element_type=jnp.float32)
    z_ref[...] = z * d[:, None]
    d_ref[...] = d[:, None]


def _tc_b(x_pad, W0, degp):
    grid = (N_PAD // _BLK,)
    return pl.pallas_call(
        _tc_b_body,
        grid=grid,
        in_specs=[
            pl.BlockSpec((_BLK, D), lambda i: (i, 0)),
            pl.BlockSpec((D, D), lambda i: (0, 0)),
            pl.BlockSpec((NCORES, _BLK, 16), lambda i: (0, i, 0)),
        ],
        out_specs=[
            pl.BlockSpec((_BLK, D), lambda i: (i, 0)),
            pl.BlockSpec((_BLK, 1), lambda i: (i, 0)),
        ],
        out_shape=[
            jax.ShapeDtypeStruct((N_PAD, D), jnp.float32),
            jax.ShapeDtypeStruct((N_PAD, 1), jnp.float32),
        ],
    )(x_pad, W0, degp)


def _tc_d_body(z_ref, p_ref, d_ref, b_ref, w_ref, z1_ref):
    s = p_ref[0] + p_ref[1] + z_ref[...]
    h = jnp.maximum(d_ref[...] * s + b_ref[...], 0.0)
    z1 = jnp.dot(h, w_ref[...], preferred_element_type=jnp.float32)
    z1_ref[...] = z1 * d_ref[...]


def _tc_d(z0, p, d, b0, W1):
    grid = (N_PAD // _BLK,)
    return pl.pallas_call(
        _tc_d_body,
        grid=grid,
        in_specs=[
            pl.BlockSpec((_BLK, D), lambda i: (i, 0)),
            pl.BlockSpec((NCORES, _BLK, D), lambda i: (0, i, 0)),
            pl.BlockSpec((_BLK, 1), lambda i: (i, 0)),
            pl.BlockSpec((1, D), lambda i: (0, 0)),
            pl.BlockSpec((D, D), lambda i: (0, 0)),
        ],
        out_specs=pl.BlockSpec((_BLK, D), lambda i: (i, 0)),
        out_shape=jax.ShapeDtypeStruct((N_PAD, D), jnp.float32),
    )(z0, p, d, b0, W1)


def _tc_e_body(z_ref, p_ref, d_ref, b_ref, wc_ref, bc_ref, o_ref):
    s = p_ref[0] + p_ref[1] + z_ref[...]
    h = jnp.maximum(d_ref[...] * s + b_ref[...], 0.0)
    logits = jnp.dot(h, wc_ref[...], preferred_element_type=jnp.float32)
    logits = logits + bc_ref[...]
    m = jnp.max(logits, axis=1, keepdims=True)
    ex = jnp.exp(logits - m)
    lse = jnp.log(jnp.sum(ex, axis=1, keepdims=True)) + m
    o_ref[...] = logits - lse


def _tc_e(z1, p, d, b1, Wc_pad, bc_pad):
    grid = (N_PAD // _BLK,)
    return pl.pallas_call(
        _tc_e_body,
        grid=grid,
        in_specs=[
            pl.BlockSpec((_BLK, D), lambda i: (i, 0)),
            pl.BlockSpec((NCORES, _BLK, D), lambda i: (0, i, 0)),
            pl.BlockSpec((_BLK, 1), lambda i: (i, 0)),
            pl.BlockSpec((1, D), lambda i: (0, 0)),
            pl.BlockSpec((D, D), lambda i: (0, 0)),
            pl.BlockSpec((1, D), lambda i: (0, 0)),
        ],
        out_specs=pl.BlockSpec((_BLK, D), lambda i: (i, 0)),
        out_shape=jax.ShapeDtypeStruct((N_PAD, D), jnp.float32),
    )(z1, p, d, b1, Wc_pad, bc_pad)


def kernel(x, edge_index, W0, b0, W1, b1, Wc, bc):
    src = edge_index[0]
    dst = edge_index[1]
    pad = E_PAD - src.shape[0]
    pad_idx = N + (jnp.arange(pad, dtype=jnp.int32) % (N_PAD - N))
    src_p = jnp.concatenate([src, pad_idx])
    dst_p = jnp.concatenate([dst, pad_idx])
    src2d = src_p.reshape(NW * CPT, CH)
    dst2d = dst_p.reshape(NW * CPT, CH)

    x_pad = jnp.zeros((N_PAD, D), jnp.float32).at[:N].set(x)
    zeros16 = jnp.zeros((N_PAD, 16), jnp.float32)
    zerosD = jnp.zeros((N_PAD, D), jnp.float32)
    b0r = b0.reshape(1, D)
    b1r = b1.reshape(1, D)
    Wc_pad = jnp.zeros((D, D), jnp.float32).at[:, :NC_OUT].set(Wc)
    bc_pad = jnp.full((1, D), -1e30, jnp.float32).at[0, :NC_OUT].set(bc)

    deg_k = _make_deg_kernel()
    scat_k = _make_scat_kernel()

    degp = deg_k(dst2d, zeros16)
    z0, d = _tc_b(x_pad, W0, degp)
    p0 = scat_k(z0, src2d, dst2d, zerosD)
    z1 = _tc_d(z0, p0, d, b0r, W1)
    p1 = scat_k(z1, src2d, dst2d, zerosD)
    full = _tc_e(z1, p1, d, b1r, Wc_pad, bc_pad)
    return full[:N, :NC_OUT]


# speedup vs baseline: 29.9952x; 1.4075x over previous
import functools

import jax
import jax.numpy as jnp
from jax import lax
from jax.experimental import pallas as pl
from jax.experimental.pallas import tpu as pltpu
from jax.experimental.pallas import tpu_sc as plsc

N = 10000
E = 320000
D = 128
NC_OUT = 40

N_PAD = 10240
CH = 128
NCORES = 2
NSUB = 16
NW = NCORES * NSUB
CPT = 80
PHC = 40
E_PAD = NW * CPT * CH
ROWS_PER_TILE = N_PAD // NSUB


def _sc_mesh():
    return plsc.VectorSubcoreMesh(core_axis_name="c", subcore_axis_name="s")


def _deg_kernel_body(dst_hbm, zeros_hbm, out_hbm, dst_v, ones_v, accd, sem):
    cid = lax.axis_index("c")
    sid = lax.axis_index("s")
    tile = cid * NSUB + sid

    def fill_ones(i, c):
        ones_v[i] = jnp.ones((16,), jnp.float32)
        return c
    lax.fori_loop(0, CH, fill_ones, 0)

    r0 = sid * ROWS_PER_TILE
    pltpu.sync_copy(zeros_hbm.at[pl.ds(r0, ROWS_PER_TILE)],
                    accd.at[pl.ds(r0, ROWS_PER_TILE)])
    plsc.subcore_barrier()

    base = tile * CPT
    pltpu.sync_copy(dst_hbm.at[pl.ds(base, CPT)], dst_v)

    def body(j, c):
        pltpu.sync_copy(ones_v, accd.at[dst_v.at[j]], add=True)
        return c
    lax.fori_loop(0, CPT, body, 0)
    plsc.subcore_barrier()

    pltpu.sync_copy(accd.at[pl.ds(r0, ROWS_PER_TILE)],
                    out_hbm.at[cid, pl.ds(r0, ROWS_PER_TILE)])


def _make_deg_kernel():
    return functools.partial(
        pl.kernel,
        mesh=_sc_mesh(),
        out_type=jax.ShapeDtypeStruct((NCORES, N_PAD, 16), jnp.float32),
        scratch_types=[
            pltpu.VMEM((CPT, CH), jnp.int32),
            pltpu.VMEM((CH, 16), jnp.float32),
            pltpu.VMEM_SHARED((N_PAD, 16), jnp.float32),
            pltpu.SemaphoreType.DMA,
        ],
    )(_deg_kernel_body)


def _scat_kernel_body(z_hbm, src_hbm, dst_hbm, zeros_hbm, out_hbm,
                      src_v, dst_v, rows_a, rows_b, accs, sem_a, sem_b):
    cid = lax.axis_index("c")
    sid = lax.axis_index("s")
    tile = cid * NSUB + sid

    r0 = sid * ROWS_PER_TILE
    pltpu.sync_copy(zeros_hbm.at[pl.ds(r0, ROWS_PER_TILE)],
                    accs.at[pl.ds(r0, ROWS_PER_TILE)])
    plsc.subcore_barrier()

    base = tile * CPT
    for phase in range(CPT // PHC):
        pbase = base + phase * PHC
        pltpu.sync_copy(src_hbm.at[pl.ds(pbase, PHC)], src_v)
        pltpu.sync_copy(dst_hbm.at[pl.ds(pbase, PHC)], dst_v)
        pltpu.async_copy(z_hbm.at[src_v.at[0]], rows_a, sem_a)

        def body(i, c):
            j = 2 * i
            pltpu.async_copy(z_hbm.at[src_v.at[j + 1]], rows_b, sem_b)
            pltpu.make_async_copy(z_hbm.at[src_v.at[j]], rows_a, sem_a).wait()
            pltpu.sync_copy(rows_a, accs.at[dst_v.at[j]], add=True)

            @pl.when(j + 2 < PHC)
            def _():
                pltpu.async_copy(z_hbm.at[src_v.at[j + 2]], rows_a, sem_a)

            pltpu.make_async_copy(z_hbm.at[src_v.at[j + 1]], rows_b, sem_b).wait()
            pltpu.sync_copy(rows_b, accs.at[dst_v.at[j + 1]], add=True)
            return c
        lax.fori_loop(0, PHC // 2, body, 0)
    plsc.subcore_barrier()

    pltpu.sync_copy(accs.at[pl.ds(r0, ROWS_PER_TILE)],
                    out_hbm.at[cid, pl.ds(r0, ROWS_PER_TILE)])


def _make_scat_kernel():
    return functools.partial(
        pl.kernel,
        mesh=_sc_mesh(),
        out_type=jax.ShapeDtypeStruct((NCORES, N_PAD, D), jnp.float32),
        scratch_types=[
            pltpu.VMEM((PHC, CH), jnp.int32),
            pltpu.VMEM((PHC, CH), jnp.int32),
            pltpu.VMEM((CH, D), jnp.float32),
            pltpu.VMEM((CH, D), jnp.float32),
            pltpu.VMEM_SHARED((N_PAD, D), jnp.float32),
            pltpu.SemaphoreType.DMA,
            pltpu.SemaphoreType.DMA,
        ],
    )(_scat_kernel_body)


_BLK = 1024


def _tc_b_body(x_ref, w_ref, degp_ref, z_ref, d_ref):
    deg = 1.0 + degp_ref[0, :, 0] + degp_ref[1, :, 0]
    d = lax.rsqrt(deg)
    z = jnp.dot(x_ref[...], w_ref[...], preferred_element_type=jnp.float32)
    z_ref[...] = z * d[:, None]
    d_ref[...] = d[:, None]


def _tc_b(x_pad, W0, degp):
    grid = (N_PAD // _BLK,)
    return pl.pallas_call(
        _tc_b_body,
        grid=grid,
        in_specs=[
            pl.BlockSpec((_BLK, D), lambda i: (i, 0)),
            pl.BlockSpec((D, D), lambda i: (0, 0)),
            pl.BlockSpec((NCORES, _BLK, 16), lambda i: (0, i, 0)),
        ],
        out_specs=[
            pl.BlockSpec((_BLK, D), lambda i: (i, 0)),
            pl.BlockSpec((_BLK, 1), lambda i: (i, 0)),
        ],
        out_shape=[
            jax.ShapeDtypeStruct((N_PAD, D), jnp.float32),
            jax.ShapeDtypeStruct((N_PAD, 1), jnp.float32),
        ],
    )(x_pad, W0, degp)


def _tc_d_body(z_ref, p_ref, d_ref, b_ref, w_ref, z1_ref):
    s = p_ref[0] + p_ref[1] + z_ref[...]
    h = jnp.maximum(d_ref[...] * s + b_ref[...], 0.0)
    z1 = jnp.dot(h, w_ref[...], preferred_element_type=jnp.float32)
    z1_ref[...] = z1 * d_ref[...]


def _tc_d(z0, p, d, b0, W1):
    grid = (N_PAD // _BLK,)
    return pl.pallas_call(
        _tc_d_body,
        grid=grid,
        in_specs=[
            pl.BlockSpec((_BLK, D), lambda i: (i, 0)),
            pl.BlockSpec((NCORES, _BLK, D), lambda i: (0, i, 0)),
            pl.BlockSpec((_BLK, 1), lambda i: (i, 0)),
            pl.BlockSpec((1, D), lambda i: (0, 0)),
            pl.BlockSpec((D, D), lambda i: (0, 0)),
        ],
        out_specs=pl.BlockSpec((_BLK, D), lambda i: (i, 0)),
        out_shape=jax.ShapeDtypeStruct((N_PAD, D), jnp.float32),
    )(z0, p, d, b0, W1)


def _tc_e_body(z_ref, p_ref, d_ref, b_ref, wc_ref, bc_ref, o_ref):
    s = p_ref[0] + p_ref[1] + z_ref[...]
    h = jnp.maximum(d_ref[...] * s + b_ref[...], 0.0)
    logits = jnp.dot(h, wc_ref[...], preferred_element_type=jnp.float32)
    logits = logits + bc_ref[...]
    m = jnp.max(logits, axis=1, keepdims=True)
    ex = jnp.exp(logits - m)
    lse = jnp.log(jnp.sum(ex, axis=1, keepdims=True)) + m
    o_ref[...] = logits - lse


def _tc_e(z1, p, d, b1, Wc_pad, bc_pad):
    grid = (N_PAD // _BLK,)
    return pl.pallas_call(
        _tc_e_body,
        grid=grid,
        in_specs=[
            pl.BlockSpec((_BLK, D), lambda i: (i, 0)),
            pl.BlockSpec((NCORES, _BLK, D), lambda i: (0, i, 0)),
            pl.BlockSpec((_BLK, 1), lambda i: (i, 0)),
            pl.BlockSpec((1, D), lambda i: (0, 0)),
            pl.BlockSpec((D, D), lambda i: (0, 0)),
            pl.BlockSpec((1, D), lambda i: (0, 0)),
        ],
        out_specs=pl.BlockSpec((_BLK, D), lambda i: (i, 0)),
        out_shape=jax.ShapeDtypeStruct((N_PAD, D), jnp.float32),
    )(z1, p, d, b1, Wc_pad, bc_pad)


def kernel(x, edge_index, W0, b0, W1, b1, Wc, bc):
    src = edge_index[0]
    dst = edge_index[1]
    pad = E_PAD - src.shape[0]
    pad_idx = N + (jnp.arange(pad, dtype=jnp.int32) % (N_PAD - N))
    src_p = jnp.concatenate([src, pad_idx])
    dst_p = jnp.concatenate([dst, pad_idx])
    src2d = src_p.reshape(NW * CPT, CH)
    dst2d = dst_p.reshape(NW * CPT, CH)

    x_pad = jnp.zeros((N_PAD, D), jnp.float32).at[:N].set(x)
    zeros16 = jnp.zeros((N_PAD, 16), jnp.float32)
    zerosD = jnp.zeros((N_PAD, D), jnp.float32)
    b0r = b0.reshape(1, D)
    b1r = b1.reshape(1, D)
    Wc_pad = jnp.zeros((D, D), jnp.float32).at[:, :NC_OUT].set(Wc)
    bc_pad = jnp.full((1, D), -1e30, jnp.float32).at[0, :NC_OUT].set(bc)

    deg_k = _make_deg_kernel()
    scat_k = _make_scat_kernel()

    degp = deg_k(dst2d, zeros16)
    z0, d = _tc_b(x_pad, W0, degp)
    p0 = scat_k(z0, src2d, dst2d, zerosD)
    z1 = _tc_d(z0, p0, d, b0r, W1)
    p1 = scat_k(z1, src2d, dst2d, zerosD)
    full = _tc_e(z1, p1, d, b1r, Wc_pad, bc_pad)
    return full[:N, :NC_OUT]
